# baseline (speedup 1.0000x reference)
import functools

import jax
import jax.numpy as jnp
from jax import lax
from jax.experimental import pallas as pl
from jax.experimental.pallas import tpu as pltpu
from jax.experimental.pallas import tpu_sc as plsc

B = 16384
D = 32
NF = 3


@functools.cache
def _build():
    info = plsc.get_sparse_core_info()
    nc, ns = info.num_cores, info.num_subcores
    nw = nc * ns
    bw = B // nw
    mesh = plsc.VectorSubcoreMesh(core_axis_name="c", subcore_axis_name="s")

    @functools.partial(
        pl.kernel,
        mesh=mesh,
        out_type=jax.ShapeDtypeStruct((B * NF * D,), jnp.float32),
        scratch_types=[
            pltpu.VMEM((bw,), jnp.int32),
            pltpu.VMEM((NF * bw * D,), jnp.float32),
            pltpu.SemaphoreType.DMA,
        ],
    )
    def cat_embed(pos_h, bet_h, top_h, wp_h, wb_h, wt_h, out_h,
                  ivm, cmb, sem):
        wid = lax.axis_index("s") * nc + lax.axis_index("c")
        base = wid * bw

        for f, (idx_h, tbl) in enumerate(
                ((pos_h, wp_h), (bet_h, wb_h), (top_h, wt_h))):
            pltpu.sync_copy(idx_h.at[pl.ds(base, bw)], ivm)

            def body(k, _, f=f, tbl=tbl):
                vec = ivm[pl.ds(k * 16, 16)]
                for l in range(16):
                    r = k * 16 + l
                    src_off = pl.multiple_of(vec[l] * D, D)
                    dst_off = pl.multiple_of((NF * r + f) * D, D)
                    pltpu.async_copy(tbl.at[pl.ds(src_off, D)],
                                     cmb.at[pl.ds(dst_off, D)], sem)
                return ()

            lax.fori_loop(0, bw // 16, body, ())

        pltpu.make_async_copy(
            out_h.at[pl.ds(NF * D * base, NF * D * bw)], cmb, sem).wait()
        pltpu.sync_copy(cmb, out_h.at[pl.ds(NF * D * base, NF * D * bw)])

    return cat_embed


def kernel(positions, bet_sizing_id, topology,
           W_positions, W_bet_sizing_id, W_topology):
    out = _build()(positions, bet_sizing_id, topology,
                   W_positions.reshape(-1), W_bet_sizing_id.reshape(-1),
                   W_topology.reshape(-1))
    return out.reshape(B, NF * D)
